# baseline (speedup 1.0000x reference)
import functools

import jax
import jax.numpy as jnp
from jax import lax
from jax.experimental import pallas as pl
from jax.experimental.pallas import tpu as pltpu
from jax.experimental.pallas import tpu_sc as plsc

EE = 16
DM = 1024
DF = 4096
NT = 4096
NP = 8192
BROW = 512
GBLK = NP // BROW + EE
LPAD = GBLK * BROW
FC = 2048
NF = DF // FC
CH = 128
NCH = NP // CH

SC_NC = 2
SC_NS = 16
NWORK = SC_NC * SC_NS
PPW = NP // NWORK
SCH = 64
NSCH = PPW // SCH

@functools.cache
def _sc_mesh():
    return plsc.VectorSubcoreMesh(core_axis_name="c", subcore_axis_name="s")


def _dispatch_body(fe_ref, pos_ref, be_ref, lv_ref, oh_s, incl_s):
    fe = fe_ref[...]
    lane = lax.broadcasted_iota(jnp.int32, (NP, EE), 1)
    oh_s[...] = (fe == lane).astype(jnp.float32)
    tri = (lax.broadcasted_iota(jnp.int32, (CH, CH), 0) >=
           lax.broadcasted_iota(jnp.int32, (CH, CH), 1)).astype(jnp.float32)

    def body(c, carry):
        blk = oh_s[pl.ds(c * CH, CH), :]
        loc = jnp.dot(tri, blk, preferred_element_type=jnp.float32)
        incl_s[pl.ds(c * CH, CH), :] = loc + carry
        return carry + jnp.sum(blk, axis=0, keepdims=True)

    counts = lax.fori_loop(0, NCH, body, jnp.zeros((1, EE), jnp.float32))
    ci = counts.astype(jnp.int32)
    pads = ((ci + (BROW - 1)) // BROW) * BROW
    pf = pads.astype(jnp.float32)
    triu = (lax.broadcasted_iota(jnp.int32, (EE, EE), 0) <
            lax.broadcasted_iota(jnp.int32, (EE, EE), 1)).astype(jnp.float32)
    pad_off = jnp.dot(pf, triu, preferred_element_type=jnp.float32)

    posf = jnp.sum(oh_s[...] * (incl_s[...] + pad_off), axis=1,
                   keepdims=True) - 1.0
    pos_ref[...] = posf.astype(jnp.int32)

    gb = (lax.broadcasted_iota(jnp.int32, (GBLK, EE), 0) * BROW).astype(jnp.float32)
    ind = jnp.logical_and(gb >= pad_off, gb < pad_off + pf).astype(jnp.float32)
    ev = lax.broadcasted_iota(jnp.int32, (GBLK, EE), 1).astype(jnp.float32)
    be_ref[...] = jnp.sum(ind * ev, axis=1, keepdims=True).astype(jnp.int32)
    lv_ref[...] = jnp.sum(ind, axis=1, keepdims=True).astype(jnp.int32)


_dispatch = pl.pallas_call(
    _dispatch_body,
    out_shape=(jax.ShapeDtypeStruct((NP, 1), jnp.int32),
               jax.ShapeDtypeStruct((GBLK, 1), jnp.int32),
               jax.ShapeDtypeStruct((GBLK, 1), jnp.int32)),
    scratch_shapes=[pltpu.VMEM((NP, EE), jnp.float32),
                    pltpu.VMEM((NP, EE), jnp.float32)],
)


def _permute_body(x_hbm, pos_hbm, sc_hbm, out_hbm, ssort_hbm,
                  posv, tokv, scv, rows, sem):
    wid = lax.axis_index("s") * SC_NC + lax.axis_index("c")
    base = wid * PPW

    def chunk(s, carry):
        pb = base + s * SCH
        pltpu.sync_copy(pos_hbm.at[pl.ds(pb, SCH)], posv)
        pltpu.sync_copy(sc_hbm.at[pl.ds(pb, SCH)], scv)
        for u in range(SCH // 16):
            tokv[pl.ds(u * 16, 16)] = lax.shift_right_logical(
                lax.iota(jnp.int32, 16) + (pb + u * 16), 1)
        pltpu.async_copy(x_hbm.at[tokv], rows, sem).wait()
        pltpu.async_copy(rows, out_hbm.at[posv], sem).wait()
        pltpu.async_copy(scv, ssort_hbm.at[posv], sem).wait()
        return carry

    lax.fori_loop(0, NSCH, chunk, 0)


@functools.cache
def _permute():
    return pl.kernel(
        _permute_body,
        mesh=_sc_mesh(),
        out_type=(jax.ShapeDtypeStruct((LPAD, DM), jnp.float32),
                  jax.ShapeDtypeStruct((LPAD,), jnp.float32)),
        scratch_types=[pltpu.VMEM((SCH,), jnp.int32),
                       pltpu.VMEM((SCH,), jnp.int32),
                       pltpu.VMEM((SCH,), jnp.float32),
                       pltpu.VMEM((SCH, DM), jnp.float32),
                       pltpu.SemaphoreType.DMA],
    )


def _gemm_body(be_ref, lv_ref, xs_ref, w1_ref, b1_ref, w2_ref, b2_ref,
               sc_ref, o_ref):
    ff = pl.program_id(1)
    g = pl.program_id(0)

    @pl.when(lv_ref[g] == 1)
    def _compute():
        xb = xs_ref[...].astype(jnp.bfloat16)
        w1 = w1_ref[0].astype(jnp.bfloat16)
        h = jnp.dot(xb, w1, preferred_element_type=jnp.float32)
        h = jax.nn.gelu(h + b1_ref[0])
        w2 = w2_ref[0].astype(jnp.bfloat16)
        contrib = jnp.dot(h.astype(jnp.bfloat16), w2,
                          preferred_element_type=jnp.float32)

        @pl.when(ff == 0)
        def _init():
            o_ref[...] = contrib + b2_ref[0]

        @pl.when(ff != 0)
        def _acc():
            o_ref[...] += contrib

        @pl.when(ff == NF - 1)
        def _scale():
            o_ref[...] *= sc_ref[...]


_gemm = pl.pallas_call(
    _gemm_body,
    grid_spec=pltpu.PrefetchScalarGridSpec(
        num_scalar_prefetch=2,
        grid=(GBLK, NF),
        in_specs=[
            pl.BlockSpec((BROW, DM), lambda g, f, be, lv: (g * lv[g], 0)),
            pl.BlockSpec((1, DM, FC),
                         lambda g, f, be, lv: (be[g], 0,
                                               (f ^ (g % 2)) * lv[g])),
            pl.BlockSpec((1, 1, FC),
                         lambda g, f, be, lv: (be[g], 0,
                                               (f ^ (g % 2)) * lv[g])),
            pl.BlockSpec((1, FC, DM),
                         lambda g, f, be, lv: (be[g],
                                               (f ^ (g % 2)) * lv[g], 0)),
            pl.BlockSpec((1, 1, DM), lambda g, f, be, lv: (be[g], 0, 0)),
            pl.BlockSpec((BROW, 1), lambda g, f, be, lv: (g, 0)),
        ],
        out_specs=pl.BlockSpec((BROW, DM), lambda g, f, be, lv: (g, 0)),
    ),
    out_shape=jax.ShapeDtypeStruct((LPAD, DM), jnp.float32),
    compiler_params=pltpu.CompilerParams(
        dimension_semantics=("arbitrary", "arbitrary")),
)


def _comb_body(o_hbm, pos_hbm, y_hbm, posv, rows, yv, sem):
    wid = lax.axis_index("s") * SC_NC + lax.axis_index("c")
    base = wid * PPW
    ntk = SCH // 2

    def chunk(s, carry):
        pb = base + s * SCH
        pltpu.sync_copy(pos_hbm.at[pl.ds(pb, SCH)], posv)
        pltpu.async_copy(o_hbm.at[posv], rows, sem).wait()

        def tok(t, c2):
            for v in range(DM // 16):
                yv[t, pl.ds(v * 16, 16)] = (rows[2 * t, pl.ds(v * 16, 16)] +
                                            rows[2 * t + 1, pl.ds(v * 16, 16)])
            return c2

        lax.fori_loop(0, ntk, tok, 0)
        tb = wid * (PPW // 2) + s * ntk
        pltpu.sync_copy(yv, y_hbm.at[pl.ds(tb, ntk)])
        return carry

    lax.fori_loop(0, NSCH, chunk, 0)


@functools.cache
def _comb():
    return pl.kernel(
        _comb_body,
        mesh=_sc_mesh(),
        out_type=jax.ShapeDtypeStruct((NT, DM), jnp.float32),
        scratch_types=[pltpu.VMEM((SCH,), jnp.int32),
                       pltpu.VMEM((SCH, DM), jnp.float32),
                       pltpu.VMEM((SCH // 2, DM), jnp.float32),
                       pltpu.SemaphoreType.DMA],
    )


def kernel(x, topK_indices, topK_scores, W1, b1, W2, b2):
    flat = topK_indices.reshape(NP, 1)
    pos2, be2, lv2 = _dispatch(flat)
    pos = pos2.reshape(NP)
    be = be2.reshape(GBLK)
    lv = lv2.reshape(GBLK)
    xs, ssort = _permute()(x, pos, topK_scores.reshape(NP))
    o = _gemm(be, lv, xs, W1, b1.reshape(EE, 1, DF), W2,
              b2.reshape(EE, 1, DM), ssort.reshape(LPAD, 1))
    y = _comb()(o, pos)
    return y


# speedup vs baseline: 1.0080x; 1.0080x over previous
import functools

import jax
import jax.numpy as jnp
from jax import lax
from jax.experimental import pallas as pl
from jax.experimental.pallas import tpu as pltpu
from jax.experimental.pallas import tpu_sc as plsc

EE = 16
DM = 1024
DF = 4096
NT = 4096
NP = 8192
BROW = 512
GBLK = NP // BROW + EE
LPAD = GBLK * BROW
FC = 2048
NF = DF // FC
CH = 128
NCH = NP // CH

SC_NC = 2
SC_NS = 16
NWORK = SC_NC * SC_NS
PPW = NP // NWORK
SCH = 64
NSCH = PPW // SCH

@functools.cache
def _sc_mesh():
    return plsc.VectorSubcoreMesh(core_axis_name="c", subcore_axis_name="s")


def _dispatch_body(fe_ref, pos_ref, be_ref, lv_ref, oh_s, incl_s):
    fe = fe_ref[...]
    lane = lax.broadcasted_iota(jnp.int32, (NP, EE), 1)
    oh_s[...] = (fe == lane).astype(jnp.float32)
    tri = (lax.broadcasted_iota(jnp.int32, (CH, CH), 0) >=
           lax.broadcasted_iota(jnp.int32, (CH, CH), 1)).astype(jnp.float32)

    def body(c, carry):
        blk = oh_s[pl.ds(c * CH, CH), :]
        loc = jnp.dot(tri, blk, preferred_element_type=jnp.float32)
        incl_s[pl.ds(c * CH, CH), :] = loc + carry
        return carry + jnp.sum(blk, axis=0, keepdims=True)

    counts = lax.fori_loop(0, NCH, body, jnp.zeros((1, EE), jnp.float32))
    ci = counts.astype(jnp.int32)
    pads = ((ci + (BROW - 1)) // BROW) * BROW
    pf = pads.astype(jnp.float32)
    triu = (lax.broadcasted_iota(jnp.int32, (EE, EE), 0) <
            lax.broadcasted_iota(jnp.int32, (EE, EE), 1)).astype(jnp.float32)
    pad_off = jnp.dot(pf, triu, preferred_element_type=jnp.float32)

    posf = jnp.sum(oh_s[...] * (incl_s[...] + pad_off), axis=1,
                   keepdims=True) - 1.0
    pos_ref[...] = posf.astype(jnp.int32)

    gb = (lax.broadcasted_iota(jnp.int32, (GBLK, EE), 0) * BROW).astype(jnp.float32)
    ind = jnp.logical_and(gb >= pad_off, gb < pad_off + pf).astype(jnp.float32)
    ev = lax.broadcasted_iota(jnp.int32, (GBLK, EE), 1).astype(jnp.float32)
    be_ref[...] = jnp.sum(ind * ev, axis=1, keepdims=True).astype(jnp.int32)
    lv_ref[...] = jnp.sum(ind, axis=1, keepdims=True).astype(jnp.int32)


_dispatch = pl.pallas_call(
    _dispatch_body,
    out_shape=(jax.ShapeDtypeStruct((NP, 1), jnp.int32),
               jax.ShapeDtypeStruct((GBLK, 1), jnp.int32),
               jax.ShapeDtypeStruct((GBLK, 1), jnp.int32)),
    scratch_shapes=[pltpu.VMEM((NP, EE), jnp.float32),
                    pltpu.VMEM((NP, EE), jnp.float32)],
)


def _permute_body(x_hbm, pos_hbm, sc_hbm, out_hbm, ssort_hbm,
                  posv, tokv, scv, rows, sem, sem2):
    wid = lax.axis_index("s") * SC_NC + lax.axis_index("c")
    base = wid * PPW

    def chunk(s, carry):
        pb = base + s * SCH
        pltpu.sync_copy(pos_hbm.at[pl.ds(pb, SCH)], posv)
        pltpu.sync_copy(sc_hbm.at[pl.ds(pb, SCH)], scv)
        for u in range(SCH // 16):
            tokv[pl.ds(u * 16, 16)] = lax.shift_right_logical(
                lax.iota(jnp.int32, 16) + (pb + u * 16), 1)
        pltpu.async_copy(x_hbm.at[tokv], rows, sem).wait()
        c1 = pltpu.async_copy(rows, out_hbm.at[posv], sem)
        c2 = pltpu.async_copy(scv, ssort_hbm.at[posv], sem2)
        c1.wait()
        c2.wait()
        return carry

    lax.fori_loop(0, NSCH, chunk, 0)


@functools.cache
def _permute():
    return pl.kernel(
        _permute_body,
        mesh=_sc_mesh(),
        out_type=(jax.ShapeDtypeStruct((LPAD, DM), jnp.float32),
                  jax.ShapeDtypeStruct((LPAD,), jnp.float32)),
        scratch_types=[pltpu.VMEM((SCH,), jnp.int32),
                       pltpu.VMEM((SCH,), jnp.int32),
                       pltpu.VMEM((SCH,), jnp.float32),
                       pltpu.VMEM((SCH, DM), jnp.float32),
                       pltpu.SemaphoreType.DMA,
                       pltpu.SemaphoreType.DMA],
    )


def _gemm_body(be_ref, lv_ref, xs_ref, w1_ref, b1_ref, w2_ref, b2_ref,
               sc_ref, o_ref):
    ff = pl.program_id(1)
    g = pl.program_id(0)

    @pl.when(lv_ref[g] == 1)
    def _compute():
        h = jnp.dot(xs_ref[...], w1_ref[0], preferred_element_type=jnp.float32)
        h = jax.nn.gelu(h + b1_ref[0])
        contrib = jnp.dot(h, w2_ref[0], preferred_element_type=jnp.float32)

        @pl.when(ff == 0)
        def _init():
            o_ref[...] = contrib + b2_ref[0]

        @pl.when(ff != 0)
        def _acc():
            o_ref[...] += contrib

        @pl.when(ff == NF - 1)
        def _scale():
            o_ref[...] *= sc_ref[...]


_gemm = pl.pallas_call(
    _gemm_body,
    grid_spec=pltpu.PrefetchScalarGridSpec(
        num_scalar_prefetch=2,
        grid=(GBLK, NF),
        in_specs=[
            pl.BlockSpec((BROW, DM), lambda g, f, be, lv: (g * lv[g], 0)),
            pl.BlockSpec((1, DM, FC),
                         lambda g, f, be, lv: (be[g], 0,
                                               f * lv[g])),
            pl.BlockSpec((1, 1, FC),
                         lambda g, f, be, lv: (be[g], 0,
                                               f * lv[g])),
            pl.BlockSpec((1, FC, DM),
                         lambda g, f, be, lv: (be[g], f * lv[g], 0)),
            pl.BlockSpec((1, 1, DM), lambda g, f, be, lv: (be[g], 0, 0)),
            pl.BlockSpec((BROW, 1), lambda g, f, be, lv: (g, 0)),
        ],
        out_specs=pl.BlockSpec((BROW, DM), lambda g, f, be, lv: (g, 0)),
    ),
    out_shape=jax.ShapeDtypeStruct((LPAD, DM), jnp.float32),
    compiler_params=pltpu.CompilerParams(
        dimension_semantics=("arbitrary", "arbitrary")),
)


def _comb_body(o_hbm, pos_hbm, y_hbm, posv, rows, yv, sem):
    wid = lax.axis_index("s") * SC_NC + lax.axis_index("c")
    base = wid * PPW
    ntk = SCH // 2

    def chunk(s, carry):
        pb = base + s * SCH
        pltpu.sync_copy(pos_hbm.at[pl.ds(pb, SCH)], posv)
        pltpu.async_copy(o_hbm.at[posv], rows, sem).wait()

        def tok(t, c2):
            for v in range(DM // 16):
                yv[t, pl.ds(v * 16, 16)] = (rows[2 * t, pl.ds(v * 16, 16)] +
                                            rows[2 * t + 1, pl.ds(v * 16, 16)])
            return c2

        lax.fori_loop(0, ntk, tok, 0)
        tb = wid * (PPW // 2) + s * ntk
        pltpu.sync_copy(yv, y_hbm.at[pl.ds(tb, ntk)])
        return carry

    lax.fori_loop(0, NSCH, chunk, 0)


@functools.cache
def _comb():
    return pl.kernel(
        _comb_body,
        mesh=_sc_mesh(),
        out_type=jax.ShapeDtypeStruct((NT, DM), jnp.float32),
        scratch_types=[pltpu.VMEM((SCH,), jnp.int32),
                       pltpu.VMEM((SCH, DM), jnp.float32),
                       pltpu.VMEM((SCH // 2, DM), jnp.float32),
                       pltpu.SemaphoreType.DMA],
    )


def kernel(x, topK_indices, topK_scores, W1, b1, W2, b2):
    flat = topK_indices.reshape(NP, 1)
    pos2, be2, lv2 = _dispatch(flat)
    pos = pos2.reshape(NP)
    be = be2.reshape(GBLK)
    lv = lv2.reshape(GBLK)
    xs, ssort = _permute()(x, pos, topK_scores.reshape(NP))
    o = _gemm(be, lv, xs, W1, b1.reshape(EE, 1, DF), W2,
              b2.reshape(EE, 1, DM), ssort.reshape(LPAD, 1))
    y = _comb()(o, pos)
    return y


# speedup vs baseline: 1.0414x; 1.0331x over previous
import functools

import jax
import jax.numpy as jnp
from jax import lax
from jax.experimental import pallas as pl
from jax.experimental.pallas import tpu as pltpu
from jax.experimental.pallas import tpu_sc as plsc

EE = 16
DM = 1024
DF = 4096
NT = 4096
NP = 8192
BROW = 512
GBLK = NP // BROW + EE
LPAD = GBLK * BROW
FC = 2048
NF = DF // FC
CH = 128
NCH = NP // CH

SC_NC = 2
SC_NS = 16
NWORK = SC_NC * SC_NS
PPW = NP // NWORK
SCH = 64
NSCH = PPW // SCH

@functools.cache
def _sc_mesh():
    return plsc.VectorSubcoreMesh(core_axis_name="c", subcore_axis_name="s")


def _dispatch_body(fe_ref, pos_ref, be_ref, lv_ref, oh_s, incl_s):
    fe = fe_ref[...]
    lane = lax.broadcasted_iota(jnp.int32, (NP, EE), 1)
    oh_s[...] = (fe == lane).astype(jnp.float32)
    tri = (lax.broadcasted_iota(jnp.int32, (CH, CH), 0) >=
           lax.broadcasted_iota(jnp.int32, (CH, CH), 1)).astype(jnp.float32)

    def body(c, carry):
        blk = oh_s[pl.ds(c * CH, CH), :]
        loc = jnp.dot(tri, blk, preferred_element_type=jnp.float32)
        incl_s[pl.ds(c * CH, CH), :] = loc + carry
        return carry + jnp.sum(blk, axis=0, keepdims=True)

    counts = lax.fori_loop(0, NCH, body, jnp.zeros((1, EE), jnp.float32))
    ci = counts.astype(jnp.int32)
    pads = ((ci + (BROW - 1)) // BROW) * BROW
    pf = pads.astype(jnp.float32)
    triu = (lax.broadcasted_iota(jnp.int32, (EE, EE), 0) <
            lax.broadcasted_iota(jnp.int32, (EE, EE), 1)).astype(jnp.float32)
    pad_off = jnp.dot(pf, triu, preferred_element_type=jnp.float32)

    posf = jnp.sum(oh_s[...] * (incl_s[...] + pad_off), axis=1,
                   keepdims=True) - 1.0
    pos_ref[...] = posf.astype(jnp.int32)

    gb = (lax.broadcasted_iota(jnp.int32, (GBLK, EE), 0) * BROW).astype(jnp.float32)
    ind = jnp.logical_and(gb >= pad_off, gb < pad_off + pf).astype(jnp.float32)
    ev = lax.broadcasted_iota(jnp.int32, (GBLK, EE), 1).astype(jnp.float32)
    be_ref[...] = jnp.sum(ind * ev, axis=1, keepdims=True).astype(jnp.int32)
    lv_ref[...] = jnp.sum(ind, axis=1, keepdims=True).astype(jnp.int32)


_dispatch = pl.pallas_call(
    _dispatch_body,
    out_shape=(jax.ShapeDtypeStruct((NP, 1), jnp.int32),
               jax.ShapeDtypeStruct((GBLK, 1), jnp.int32),
               jax.ShapeDtypeStruct((GBLK, 1), jnp.int32)),
    scratch_shapes=[pltpu.VMEM((NP, EE), jnp.float32),
                    pltpu.VMEM((NP, EE), jnp.float32)],
)


def _permute_body(x_hbm, pos_hbm, out_hbm, posv, tokv, rows, sem):
    wid = lax.axis_index("s") * SC_NC + lax.axis_index("c")
    base = wid * PPW

    def chunk(s, carry):
        pb = base + s * SCH
        pltpu.sync_copy(pos_hbm.at[pl.ds(pb, SCH)], posv)
        for u in range(SCH // 16):
            tokv[pl.ds(u * 16, 16)] = lax.shift_right_logical(
                lax.iota(jnp.int32, 16) + (pb + u * 16), 1)
        pltpu.async_copy(x_hbm.at[tokv], rows, sem).wait()
        pltpu.async_copy(rows, out_hbm.at[posv], sem).wait()
        return carry

    lax.fori_loop(0, NSCH, chunk, 0)


@functools.cache
def _permute():
    return pl.kernel(
        _permute_body,
        mesh=_sc_mesh(),
        out_type=jax.ShapeDtypeStruct((LPAD, DM), jnp.float32),
        scratch_types=[pltpu.VMEM((SCH,), jnp.int32),
                       pltpu.VMEM((SCH,), jnp.int32),
                       pltpu.VMEM((SCH, DM), jnp.float32),
                       pltpu.SemaphoreType.DMA],
    )


def _gemm_body(be_ref, lv_ref, xs_ref, w1_ref, b1_ref, w2_ref, b2_ref, o_ref):
    ff = pl.program_id(1)
    g = pl.program_id(0)

    @pl.when(lv_ref[g] == 1)
    def _compute():
        h = jnp.dot(xs_ref[...], w1_ref[0], preferred_element_type=jnp.float32)
        h = jax.nn.gelu(h + b1_ref[0])
        contrib = jnp.dot(h, w2_ref[0], preferred_element_type=jnp.float32)

        @pl.when(ff == 0)
        def _init():
            o_ref[...] = contrib + b2_ref[0]

        @pl.when(ff != 0)
        def _acc():
            o_ref[...] += contrib


_gemm = pl.pallas_call(
    _gemm_body,
    grid_spec=pltpu.PrefetchScalarGridSpec(
        num_scalar_prefetch=2,
        grid=(GBLK, NF),
        in_specs=[
            pl.BlockSpec((BROW, DM), lambda g, f, be, lv: (g * lv[g], 0)),
            pl.BlockSpec((1, DM, FC),
                         lambda g, f, be, lv: (be[g], 0, f * lv[g])),
            pl.BlockSpec((1, 1, FC),
                         lambda g, f, be, lv: (be[g], 0, f * lv[g])),
            pl.BlockSpec((1, FC, DM),
                         lambda g, f, be, lv: (be[g], f * lv[g], 0)),
            pl.BlockSpec((1, 1, DM), lambda g, f, be, lv: (be[g], 0, 0)),
        ],
        out_specs=pl.BlockSpec((BROW, DM), lambda g, f, be, lv: (g, 0)),
    ),
    out_shape=jax.ShapeDtypeStruct((LPAD, DM), jnp.float32),
    compiler_params=pltpu.CompilerParams(
        dimension_semantics=("arbitrary", "arbitrary")),
)


def _gather_body(o_hbm, pos_hbm, out_hbm, posv, rows, sem):
    wid = lax.axis_index("s") * SC_NC + lax.axis_index("c")
    base = wid * PPW

    def chunk(s, carry):
        pb = base + s * SCH
        pltpu.sync_copy(pos_hbm.at[pl.ds(pb, SCH)], posv)
        pltpu.async_copy(o_hbm.at[posv], rows, sem).wait()
        pltpu.sync_copy(rows, out_hbm.at[pl.ds(pb, SCH)])
        return carry

    lax.fori_loop(0, NSCH, chunk, 0)


@functools.cache
def _gather():
    return pl.kernel(
        _gather_body,
        mesh=_sc_mesh(),
        out_type=jax.ShapeDtypeStruct((NP, DM), jnp.float32),
        scratch_types=[pltpu.VMEM((SCH,), jnp.int32),
                       pltpu.VMEM((SCH, DM), jnp.float32),
                       pltpu.SemaphoreType.DMA],
    )


BT = 256


def _combine_body(g_ref, s0_ref, s1_ref, y_ref):
    y_ref[...] = (g_ref[:, :DM] * s0_ref[...] +
                  g_ref[:, DM:] * s1_ref[...])


_combine = pl.pallas_call(
    _combine_body,
    grid=(NT // BT,),
    in_specs=[
        pl.BlockSpec((BT, 2 * DM), lambda i: (i, 0)),
        pl.BlockSpec((BT, 1), lambda i: (i, 0)),
        pl.BlockSpec((BT, 1), lambda i: (i, 0)),
    ],
    out_specs=pl.BlockSpec((BT, DM), lambda i: (i, 0)),
    out_shape=jax.ShapeDtypeStruct((NT, DM), jnp.float32),
)


def kernel(x, topK_indices, topK_scores, W1, b1, W2, b2):
    flat = topK_indices.reshape(NP, 1)
    pos2, be2, lv2 = _dispatch(flat)
    pos = pos2.reshape(NP)
    be = be2.reshape(GBLK)
    lv = lv2.reshape(GBLK)
    xs = _permute()(x, pos)
    o = _gemm(be, lv, xs, W1, b1.reshape(EE, 1, DF), W2, b2.reshape(EE, 1, DM))
    gath = _gather()(o, pos)
    y = _combine(gath.reshape(NT, 2 * DM),
                 topK_scores[:, :1], topK_scores[:, 1:])
    return y
